# Initial kernel scaffold; baseline (speedup 1.0000x reference)
import jax
import jax.numpy as jnp
from jax.experimental import pallas as pl


def kernel(queries, roi_features, roi_mask, sa_w_in, sa_b_in, sa_w_out, sa_b_out, ln1_g, ln1_b, ca_w_in, ca_b_in, ca_w_out, ca_b_out, ln2_g, ln2_b, ffn_w1, ffn_b1, ffn_w2, ffn_b2, ln3_g, ln3_b):
    raise NotImplementedError("write your pallas kernel here")


# speedup vs baseline: 1.1548x; 1.1548x over previous
import functools
import math

import jax
import jax.numpy as jnp
from jax.experimental import pallas as pl

NHEADS = 16


def _layernorm(x, g, b, eps=1e-5):
    mu = jnp.mean(x, axis=-1, keepdims=True)
    var = jnp.mean((x - mu) ** 2, axis=-1, keepdims=True)
    return (x - mu) / jnp.sqrt(var + eps) * g + b


def _sa_kernel(q_ref, w_in_t_ref, b_in_ref, w_out_t_ref, b_out_ref,
               g1_ref, b1_ref, o_ref, *, n, d, nh):
    dh = d // nh
    x = q_ref[0]
    qkv = jnp.dot(x, w_in_t_ref[...]) + b_in_ref[...]
    qh = qkv[:, :d].reshape(n, nh, dh)
    kh = qkv[:, d:2 * d].reshape(n, nh, dh)
    vh = qkv[:, 2 * d:].reshape(n, nh, dh)
    s = jnp.einsum('qhd,khd->hqk', qh, kh) * (1.0 / math.sqrt(dh))
    m = jnp.max(s, axis=-1, keepdims=True)
    e = jnp.exp(s - m)
    a = e / jnp.sum(e, axis=-1, keepdims=True)
    o = jnp.einsum('hqk,khd->qhd', a, vh).reshape(n, d)
    o = jnp.dot(o, w_out_t_ref[...]) + b_out_ref[...]
    o_ref[0] = _layernorm(x + o, g1_ref[...], b1_ref[...])


def _ca_kernel(x_ref, rf_ref, madd_ref, wq_t_ref, bq_ref, wk_h_ref, bk_h_ref,
               wv_h_ref, bv_h_ref, w_out_t_ref, b_out_ref, g2_ref, b2_ref,
               o_ref, *, d, nh):
    dh = d // nh
    x = x_ref[...]
    g = x.shape[0]
    rf = rf_ref[...]
    q = jnp.dot(x, wq_t_ref[...]) + bq_ref[...]
    qh = q.reshape(g, nh, dh)
    qt = jnp.einsum('ghd,hde->ghe', qh, wk_h_ref[...])
    skq = jnp.einsum('ghd,hd->gh', qh, bk_h_ref[...])
    s = (jnp.einsum('ghe,gle->ghl', qt, rf) + skq[:, :, None]) * (1.0 / math.sqrt(dh))
    s = s + madd_ref[...][:, None, :]
    m = jnp.max(s, axis=-1, keepdims=True)
    e = jnp.exp(s - m)
    a = e / jnp.sum(e, axis=-1, keepdims=True)
    z = jnp.einsum('ghl,gle->ghe', a, rf)
    ov = jnp.einsum('ghe,hke->ghk', z, wv_h_ref[...]) + bv_h_ref[...]
    o = jnp.dot(ov.reshape(g, d), w_out_t_ref[...]) + b_out_ref[...]
    o_ref[...] = _layernorm(x + o, g2_ref[...], b2_ref[...])


def _ffn_kernel(x_ref, w1_t_ref, b1_ref, w2_t_ref, b2_ref, g3_ref, b3_ref, o_ref):
    x = x_ref[...]
    h = jnp.dot(x, w1_t_ref[...]) + b1_ref[...]
    h = 0.5 * h * (1.0 + jax.lax.erf(h * (1.0 / math.sqrt(2.0))))
    h = jnp.dot(h, w2_t_ref[...]) + b2_ref[...]
    o_ref[...] = _layernorm(x + h, g3_ref[...], b3_ref[...])


def kernel(queries, roi_features, roi_mask, sa_w_in, sa_b_in, sa_w_out, sa_b_out,
           ln1_g, ln1_b, ca_w_in, ca_b_in, ca_w_out, ca_b_out, ln2_g, ln2_b,
           ffn_w1, ffn_b1, ffn_w2, ffn_b2, ln3_g, ln3_b):
    B, N, D = queries.shape
    L = roi_features.shape[2]
    nh = NHEADS
    dh = D // nh
    f32 = jnp.float32

    row = lambda v: v.reshape(1, -1)
    rep2 = lambda shape: pl.BlockSpec(shape, lambda i: (0, 0))
    rep3 = lambda shape: pl.BlockSpec(shape, lambda i: (0, 0, 0))

    x1 = pl.pallas_call(
        functools.partial(_sa_kernel, n=N, d=D, nh=nh),
        grid=(B,),
        in_specs=[
            pl.BlockSpec((1, N, D), lambda i: (i, 0, 0)),
            rep2((D, 3 * D)), rep2((1, 3 * D)),
            rep2((D, D)), rep2((1, D)),
            rep2((1, D)), rep2((1, D)),
        ],
        out_specs=pl.BlockSpec((1, N, D), lambda i: (i, 0, 0)),
        out_shape=jax.ShapeDtypeStruct((B, N, D), f32),
    )(queries, sa_w_in.T, row(sa_b_in), sa_w_out.T, row(sa_b_out),
      row(ln1_g), row(ln1_b))

    S = B * N
    G = 16
    xs = x1.reshape(S, D)
    rf = roi_features.reshape(S, L, D)
    madd = jnp.where(roi_mask.reshape(S, L), jnp.float32(-1e9), jnp.float32(0.0))
    wq_t = ca_w_in[:D].T
    wk_h = ca_w_in[D:2 * D].reshape(nh, dh, D)
    wv_h = ca_w_in[2 * D:].reshape(nh, dh, D)
    bq = row(ca_b_in[:D])
    bk_h = ca_b_in[D:2 * D].reshape(nh, dh)
    bv_h = ca_b_in[2 * D:].reshape(nh, dh)

    x2 = pl.pallas_call(
        functools.partial(_ca_kernel, d=D, nh=nh),
        grid=(S // G,),
        in_specs=[
            pl.BlockSpec((G, D), lambda i: (i, 0)),
            pl.BlockSpec((G, L, D), lambda i: (i, 0, 0)),
            pl.BlockSpec((G, L), lambda i: (i, 0)),
            rep2((D, D)), rep2((1, D)),
            rep3((nh, dh, D)), rep2((nh, dh)),
            rep3((nh, dh, D)), rep2((nh, dh)),
            rep2((D, D)), rep2((1, D)),
            rep2((1, D)), rep2((1, D)),
        ],
        out_specs=pl.BlockSpec((G, D), lambda i: (i, 0)),
        out_shape=jax.ShapeDtypeStruct((S, D), f32),
    )(xs, rf, madd, wq_t, bq, wk_h, bk_h, wv_h, bv_h,
      ca_w_out.T, row(ca_b_out), row(ln2_g), row(ln2_b))

    F = ffn_w1.shape[0]
    M = 200
    out = pl.pallas_call(
        _ffn_kernel,
        grid=(S // M,),
        in_specs=[
            pl.BlockSpec((M, D), lambda i: (i, 0)),
            rep2((D, F)), rep2((1, F)),
            rep2((F, D)), rep2((1, D)),
            rep2((1, D)), rep2((1, D)),
        ],
        out_specs=pl.BlockSpec((M, D), lambda i: (i, 0)),
        out_shape=jax.ShapeDtypeStruct((S, D), f32),
    )(x2, ffn_w1.T, row(ffn_b1), ffn_w2.T, row(ffn_b2), row(ln3_g), row(ln3_b))

    return out.reshape(B, N, D)


# speedup vs baseline: 1.6096x; 1.3939x over previous
import functools
import math

import jax
import jax.numpy as jnp
from jax.experimental import pallas as pl

NHEADS = 16


def _layernorm(x, g, b, eps=1e-5):
    mu = jnp.mean(x, axis=-1, keepdims=True)
    var = jnp.mean((x - mu) ** 2, axis=-1, keepdims=True)
    return (x - mu) / jnp.sqrt(var + eps) * g + b


def _sa_kernel(q_ref, w_in_t_ref, b_in_ref, w_out_t_ref, b_out_ref,
               g1_ref, b1_ref, o_ref, *, n, d, nh):
    dh = d // nh
    x = q_ref[0]
    qkv = jnp.dot(x, w_in_t_ref[...]) + b_in_ref[...]
    qh = qkv[:, :d].reshape(n, nh, dh)
    kh = qkv[:, d:2 * d].reshape(n, nh, dh)
    vh = qkv[:, 2 * d:].reshape(n, nh, dh)
    s = jnp.einsum('qhd,khd->hqk', qh, kh) * (1.0 / math.sqrt(dh))
    m = jnp.max(s, axis=-1, keepdims=True)
    e = jnp.exp(s - m)
    a = e / jnp.sum(e, axis=-1, keepdims=True)
    o = jnp.einsum('hqk,khd->qhd', a, vh).reshape(n, d)
    o = jnp.dot(o, w_out_t_ref[...]) + b_out_ref[...]
    o_ref[0] = _layernorm(x + o, g1_ref[...], b1_ref[...])


def _ca_kernel(x_ref, rf_ref, madd_ref, wq_t_ref, bq_ref, wk_ref,
               wv_t_ref, bv_h_ref, w_out_t_ref, b_out_ref, g2_ref, b2_ref,
               o_ref, *, d, nh):
    dh = d // nh
    x = x_ref[...]
    g = x.shape[0]
    rf = rf_ref[...]
    q = jnp.dot(x, wq_t_ref[...]) + bq_ref[...]
    lane = jax.lax.broadcasted_iota(jnp.int32, (nh, d), 1)
    head = jax.lax.broadcasted_iota(jnp.int32, (nh, d), 0)
    hmask = (lane // dh == head).astype(q.dtype)
    bd = (q[:, None, :] * hmask[None, :, :]).reshape(g * nh, d)
    qt = jnp.dot(bd, wk_ref[...]).reshape(g, nh, d)
    s = jnp.einsum('ghe,gle->ghl', qt, rf) * (1.0 / math.sqrt(dh))
    s = s + madd_ref[...][:, None, :]
    m = jnp.max(s, axis=-1, keepdims=True)
    e = jnp.exp(s - m)
    a = e / jnp.sum(e, axis=-1, keepdims=True)
    z = jnp.einsum('ghl,gle->ghe', a, rf)
    ov = jnp.einsum('ghe,hek->ghk', z, wv_t_ref[...]) + bv_h_ref[...]
    o = jnp.dot(ov.reshape(g, d), w_out_t_ref[...]) + b_out_ref[...]
    o_ref[...] = _layernorm(x + o, g2_ref[...], b2_ref[...])


def _ffn_kernel(x_ref, w1_t_ref, b1_ref, w2_t_ref, b2_ref, g3_ref, b3_ref, o_ref):
    x = x_ref[...]
    h = jnp.dot(x, w1_t_ref[...]) + b1_ref[...]
    h = 0.5 * h * (1.0 + jax.lax.erf(h * (1.0 / math.sqrt(2.0))))
    h = jnp.dot(h, w2_t_ref[...]) + b2_ref[...]
    o_ref[...] = _layernorm(x + h, g3_ref[...], b3_ref[...])


def kernel(queries, roi_features, roi_mask, sa_w_in, sa_b_in, sa_w_out, sa_b_out,
           ln1_g, ln1_b, ca_w_in, ca_b_in, ca_w_out, ca_b_out, ln2_g, ln2_b,
           ffn_w1, ffn_b1, ffn_w2, ffn_b2, ln3_g, ln3_b):
    B, N, D = queries.shape
    L = roi_features.shape[2]
    nh = NHEADS
    dh = D // nh
    f32 = jnp.float32

    row = lambda v: v.reshape(1, -1)
    rep2 = lambda shape: pl.BlockSpec(shape, lambda i: (0, 0))
    rep3 = lambda shape: pl.BlockSpec(shape, lambda i: (0, 0, 0))

    x1 = pl.pallas_call(
        functools.partial(_sa_kernel, n=N, d=D, nh=nh),
        grid=(B,),
        in_specs=[
            pl.BlockSpec((1, N, D), lambda i: (i, 0, 0)),
            rep2((D, 3 * D)), rep2((1, 3 * D)),
            rep2((D, D)), rep2((1, D)),
            rep2((1, D)), rep2((1, D)),
        ],
        out_specs=pl.BlockSpec((1, N, D), lambda i: (i, 0, 0)),
        out_shape=jax.ShapeDtypeStruct((B, N, D), f32),
    )(queries, sa_w_in.T, row(sa_b_in), sa_w_out.T, row(sa_b_out),
      row(ln1_g), row(ln1_b))

    S = B * N
    G = 32
    xs = x1.reshape(S, D)
    rf = roi_features.reshape(S, L, D)
    madd = jnp.where(roi_mask.reshape(S, L), jnp.float32(-1e9), jnp.float32(0.0))
    wq_t = ca_w_in[:D].T
    wk_mat = ca_w_in[D:2 * D]
    wv_t = ca_w_in[2 * D:].reshape(nh, dh, D).transpose(0, 2, 1)
    bq = row(ca_b_in[:D])
    bv_h = ca_b_in[2 * D:].reshape(nh, dh)

    x2 = pl.pallas_call(
        functools.partial(_ca_kernel, d=D, nh=nh),
        grid=(S // G,),
        in_specs=[
            pl.BlockSpec((G, D), lambda i: (i, 0)),
            pl.BlockSpec((G, L, D), lambda i: (i, 0, 0)),
            pl.BlockSpec((G, L), lambda i: (i, 0)),
            rep2((D, D)), rep2((1, D)),
            rep2((D, D)),
            rep3((nh, D, dh)), rep2((nh, dh)),
            rep2((D, D)), rep2((1, D)),
            rep2((1, D)), rep2((1, D)),
        ],
        out_specs=pl.BlockSpec((G, D), lambda i: (i, 0)),
        out_shape=jax.ShapeDtypeStruct((S, D), f32),
    )(xs, rf, madd, wq_t, bq, wk_mat, wv_t, bv_h,
      ca_w_out.T, row(ca_b_out), row(ln2_g), row(ln2_b))

    F = ffn_w1.shape[0]
    M = 200
    out = pl.pallas_call(
        _ffn_kernel,
        grid=(S // M,),
        in_specs=[
            pl.BlockSpec((M, D), lambda i: (i, 0)),
            rep2((D, F)), rep2((1, F)),
            rep2((F, D)), rep2((1, D)),
            rep2((1, D)), rep2((1, D)),
        ],
        out_specs=pl.BlockSpec((M, D), lambda i: (i, 0)),
        out_shape=jax.ShapeDtypeStruct((S, D), f32),
    )(x2, ffn_w1.T, row(ffn_b1), ffn_w2.T, row(ffn_b2), row(ln3_g), row(ln3_b))

    return out.reshape(B, N, D)


# speedup vs baseline: 3.0525x; 1.8964x over previous
import functools
import math

import jax
import jax.numpy as jnp
from jax.experimental import pallas as pl

NHEADS = 16


def _layernorm(x, g, b, eps=1e-5):
    mu = jnp.mean(x, axis=-1, keepdims=True)
    var = jnp.mean((x - mu) ** 2, axis=-1, keepdims=True)
    return (x - mu) / jnp.sqrt(var + eps) * g + b


def _sa_kernel(q_ref, w_in_t_ref, b_in_ref, w_out_t_ref, b_out_ref,
               g1_ref, b1_ref, o_ref, *, n, d, nh):
    dh = d // nh
    x = q_ref[0]
    qkv = jnp.dot(x, w_in_t_ref[...]) + b_in_ref[...]
    qh = qkv[:, :d].reshape(n, nh, dh)
    kh = qkv[:, d:2 * d].reshape(n, nh, dh)
    vh = qkv[:, 2 * d:].reshape(n, nh, dh)
    s = jnp.einsum('qhd,khd->hqk', qh, kh) * (1.0 / math.sqrt(dh))
    m = jnp.max(s, axis=-1, keepdims=True)
    e = jnp.exp(s - m)
    a = e / jnp.sum(e, axis=-1, keepdims=True)
    o = jnp.einsum('hqk,khd->qhd', a, vh).reshape(n, d)
    o = jnp.dot(o, w_out_t_ref[...]) + b_out_ref[...]
    o_ref[0] = _layernorm(x + o, g1_ref[...], b1_ref[...])


def _ca_kernel(x_ref, rf_ref, madd_ref, wq_t_ref, bq_ref, wk_ref,
               wv_t_ref, bv_ref, w_out_t_ref, b_out_ref, g2_ref, b2_ref,
               o_ref, *, d, nh):
    dh = d // nh
    x = x_ref[...]
    g = x.shape[0]
    rf = rf_ref[...]
    q = jnp.dot(x, wq_t_ref[...]) + bq_ref[...]
    lane = jax.lax.broadcasted_iota(jnp.int32, (nh, d), 1)
    head = jax.lax.broadcasted_iota(jnp.int32, (nh, d), 0)
    hmask = (lane // dh == head).astype(q.dtype)
    bd = (q[:, None, :] * hmask[None, :, :]).reshape(g * nh, d)
    qt = jnp.dot(bd, wk_ref[...]).reshape(g, nh, d)
    s = jnp.einsum('ghe,gle->ghl', qt, rf) * (1.0 / math.sqrt(dh))
    s = s + madd_ref[...][:, None, :]
    m = jnp.max(s, axis=-1, keepdims=True)
    e = jnp.exp(s - m)
    a = e / jnp.sum(e, axis=-1, keepdims=True)
    z = jnp.einsum('ghl,gle->ghe', a, rf)
    ov_all = jnp.dot(z.reshape(g * nh, d), wv_t_ref[...]).reshape(g, nh, d)
    ov = jnp.sum(ov_all * hmask[None, :, :], axis=1) + bv_ref[...]
    o = jnp.dot(ov, w_out_t_ref[...]) + b_out_ref[...]
    o_ref[...] = _layernorm(x + o, g2_ref[...], b2_ref[...])


def _ffn_kernel(x_ref, w1_t_ref, b1_ref, w2_t_ref, b2_ref, g3_ref, b3_ref, o_ref):
    x = x_ref[...]
    h = jnp.dot(x, w1_t_ref[...]) + b1_ref[...]
    h = 0.5 * h * (1.0 + jax.lax.erf(h * (1.0 / math.sqrt(2.0))))
    h = jnp.dot(h, w2_t_ref[...]) + b2_ref[...]
    o_ref[...] = _layernorm(x + h, g3_ref[...], b3_ref[...])


def kernel(queries, roi_features, roi_mask, sa_w_in, sa_b_in, sa_w_out, sa_b_out,
           ln1_g, ln1_b, ca_w_in, ca_b_in, ca_w_out, ca_b_out, ln2_g, ln2_b,
           ffn_w1, ffn_b1, ffn_w2, ffn_b2, ln3_g, ln3_b):
    B, N, D = queries.shape
    L = roi_features.shape[2]
    nh = NHEADS
    dh = D // nh
    f32 = jnp.float32

    row = lambda v: v.reshape(1, -1)
    rep2 = lambda shape: pl.BlockSpec(shape, lambda i: (0, 0))
    rep3 = lambda shape: pl.BlockSpec(shape, lambda i: (0, 0, 0))

    x1 = pl.pallas_call(
        functools.partial(_sa_kernel, n=N, d=D, nh=nh),
        grid=(B,),
        in_specs=[
            pl.BlockSpec((1, N, D), lambda i: (i, 0, 0)),
            rep2((D, 3 * D)), rep2((1, 3 * D)),
            rep2((D, D)), rep2((1, D)),
            rep2((1, D)), rep2((1, D)),
        ],
        out_specs=pl.BlockSpec((1, N, D), lambda i: (i, 0, 0)),
        out_shape=jax.ShapeDtypeStruct((B, N, D), f32),
    )(queries, sa_w_in.T, row(sa_b_in), sa_w_out.T, row(sa_b_out),
      row(ln1_g), row(ln1_b))

    S = B * N
    G = 32
    xs = x1.reshape(S, D)
    rf = roi_features.reshape(S, L, D)
    madd = jnp.where(roi_mask.reshape(S, L), jnp.float32(-1e9), jnp.float32(0.0))
    wq_t = ca_w_in[:D].T
    wk_mat = ca_w_in[D:2 * D]
    wv_t = ca_w_in[2 * D:].T
    bq = row(ca_b_in[:D])
    bv = row(ca_b_in[2 * D:])

    x2 = pl.pallas_call(
        functools.partial(_ca_kernel, d=D, nh=nh),
        grid=(S // G,),
        in_specs=[
            pl.BlockSpec((G, D), lambda i: (i, 0)),
            pl.BlockSpec((G, L, D), lambda i: (i, 0, 0)),
            pl.BlockSpec((G, L), lambda i: (i, 0)),
            rep2((D, D)), rep2((1, D)),
            rep2((D, D)),
            rep2((D, D)), rep2((1, D)),
            rep2((D, D)), rep2((1, D)),
            rep2((1, D)), rep2((1, D)),
        ],
        out_specs=pl.BlockSpec((G, D), lambda i: (i, 0)),
        out_shape=jax.ShapeDtypeStruct((S, D), f32),
    )(xs, rf, madd, wq_t, bq, wk_mat, wv_t, bv,
      ca_w_out.T, row(ca_b_out), row(ln2_g), row(ln2_b))

    F = ffn_w1.shape[0]
    M = 200
    out = pl.pallas_call(
        _ffn_kernel,
        grid=(S // M,),
        in_specs=[
            pl.BlockSpec((M, D), lambda i: (i, 0)),
            rep2((D, F)), rep2((1, F)),
            rep2((F, D)), rep2((1, D)),
            rep2((1, D)), rep2((1, D)),
        ],
        out_specs=pl.BlockSpec((M, D), lambda i: (i, 0)),
        out_shape=jax.ShapeDtypeStruct((S, D), f32),
    )(x2, ffn_w1.T, row(ffn_b1), ffn_w2.T, row(ffn_b2), row(ln3_g), row(ln3_b))

    return out.reshape(B, N, D)
